# Initial kernel scaffold; baseline (speedup 1.0000x reference)
import jax
import jax.numpy as jnp
from jax.experimental import pallas as pl


def kernel(x, edge_index, W1, b1, W2, b2, W3, b3, Wr, br, pos):
    raise NotImplementedError("write your pallas kernel here")


# speedup vs baseline: 3.4051x; 3.4051x over previous
import functools

import jax
import jax.numpy as jnp
from jax import lax
from jax.experimental import pallas as pl
from jax.experimental.pallas import tpu as pltpu
from jax.experimental.pallas import tpu_sc as plsc

N = 10000
D = 128
E = 320000
OUT = 10

NC = 2
NS = 16
NW = NC * NS

CH = 128
K = 80
EPT = K * CH
EPAD = NW * EPT
NPAD = 10240
ROWS_PER_TILE = NPAD // NS

def _sc_aggregate_body(h_hbm, srcs_hbm, dsts_hbm, zeros_hbm, out_hbm,
                       dst_v, buf0, buf1, sidx0, sidx1, acc,
                       sem0, sem1, isem0, isem1):
    cid = lax.axis_index("c")
    sid = lax.axis_index("s")
    wid = cid * NS + sid

    pltpu.sync_copy(dsts_hbm.at[wid], dst_v)
    pltpu.sync_copy(zeros_hbm, acc.at[pl.ds(sid * ROWS_PER_TILE, ROWS_PER_TILE)])
    plsc.subcore_barrier()

    bufs = (buf0, buf1)
    sems = (sem0, sem1)
    sidxs = (sidx0, sidx1)
    isems = (isem0, isem1)

    pltpu.sync_copy(srcs_hbm.at[wid, 0], sidx0)
    pltpu.sync_copy(srcs_hbm.at[wid, 1], sidx1)
    pltpu.async_copy(h_hbm.at[sidx0], buf0, sem0)
    pltpu.async_copy(h_hbm.at[sidx1], buf1, sem1)

    def body(i, carry):
        j0 = i * 2
        for b in range(2):
            j = j0 + b
            pltpu.make_async_copy(h_hbm.at[sidxs[b]], bufs[b], sems[b]).wait()

            @pl.when(j + 2 < K)
            def _():
                pltpu.async_copy(srcs_hbm.at[wid, j + 2], sidxs[b], isems[b])

            pltpu.sync_copy(bufs[b], acc.at[dst_v.at[j]], add=True)

            @pl.when(j + 2 < K)
            def _():
                pltpu.make_async_copy(
                    srcs_hbm.at[wid, j + 2], sidxs[b], isems[b]).wait()
                pltpu.async_copy(h_hbm.at[sidxs[b]], bufs[b], sems[b])
        return carry

    lax.fori_loop(0, K // 2, body, 0)

    plsc.subcore_barrier()
    pltpu.sync_copy(acc.at[pl.ds(sid * ROWS_PER_TILE, ROWS_PER_TILE)],
                    out_hbm.at[cid, pl.ds(sid * ROWS_PER_TILE, ROWS_PER_TILE)])


@functools.cache
def _sc_aggregate():
    mesh = plsc.VectorSubcoreMesh(core_axis_name="c", subcore_axis_name="s")
    return pl.kernel(
        _sc_aggregate_body,
        mesh=mesh,
        out_type=jax.ShapeDtypeStruct((NC, NPAD, D), jnp.float32),
        scratch_types=[
            pltpu.VMEM((K, CH), jnp.int32),
            pltpu.VMEM((CH, D), jnp.float32),
            pltpu.VMEM((CH, D), jnp.float32),
            pltpu.VMEM((CH,), jnp.int32),
            pltpu.VMEM((CH,), jnp.int32),
            pltpu.VMEM_SHARED((NPAD, D), jnp.float32),
            pltpu.SemaphoreType.DMA,
            pltpu.SemaphoreType.DMA,
            pltpu.SemaphoreType.DMA,
            pltpu.SemaphoreType.DMA,
        ],
    )


_BR = 1024


def _layer_body(acc_ref, w_ref, b_ref, o_ref):
    s = acc_ref[0] + acc_ref[1]
    o_ref[...] = jnp.tanh(
        jnp.dot(s, w_ref[...], preferred_element_type=jnp.float32) + b_ref[...])


def _tc_layer(acc, W, b):
    return pl.pallas_call(
        _layer_body,
        grid=(NPAD // _BR,),
        in_specs=[
            pl.BlockSpec((NC, _BR, D), lambda i: (0, i, 0)),
            pl.BlockSpec((D, D), lambda i: (0, 0)),
            pl.BlockSpec((1, D), lambda i: (0, 0)),
        ],
        out_specs=pl.BlockSpec((_BR, D), lambda i: (i, 0)),
        out_shape=jax.ShapeDtypeStruct((NPAD, D), jnp.float32),
    )(acc, W, b.reshape(1, D))


def _final_body(acc_ref, w_ref, b_ref, wr_ref, br_ref, o_ref, sum_ref):
    i = pl.program_id(0)

    @pl.when(i == 0)
    def _():
        sum_ref[...] = jnp.zeros_like(sum_ref)

    s = acc_ref[0] + acc_ref[1]
    t = jnp.tanh(
        jnp.dot(s, w_ref[...], preferred_element_type=jnp.float32) + b_ref[...])
    rows = lax.broadcasted_iota(jnp.int32, (_BR, 1), 0) + i * _BR
    t = jnp.where(rows < N, t, 0.0)
    sum_ref[...] += jnp.sum(t, axis=0, keepdims=True)
    m = sum_ref[...] * (1.0 / N)
    o_ref[...] = jnp.tanh(
        jnp.dot(m, wr_ref[...], preferred_element_type=jnp.float32) + br_ref[...])


def _tc_final(acc, W, b, Wr_pad, br_pad):
    return pl.pallas_call(
        _final_body,
        grid=(NPAD // _BR,),
        in_specs=[
            pl.BlockSpec((NC, _BR, D), lambda i: (0, i, 0)),
            pl.BlockSpec((D, D), lambda i: (0, 0)),
            pl.BlockSpec((1, D), lambda i: (0, 0)),
            pl.BlockSpec((D, D), lambda i: (0, 0)),
            pl.BlockSpec((1, D), lambda i: (0, 0)),
        ],
        out_specs=pl.BlockSpec((1, D), lambda i: (0, 0)),
        out_shape=jax.ShapeDtypeStruct((1, D), jnp.float32),
        scratch_shapes=[pltpu.VMEM((1, D), jnp.float32)],
    )(acc, W, b.reshape(1, D), Wr_pad, br_pad)


def kernel(x, edge_index, W1, b1, W2, b2, W3, b3, Wr, br, pos):
    del pos
    src = edge_index[0]
    dst = edge_index[1]
    pad_e = EPAD - E
    srcs = jnp.concatenate([src, jnp.zeros((pad_e,), jnp.int32)]).reshape(NW, K, CH)
    dsts = jnp.concatenate([dst, jnp.full((pad_e,), N, jnp.int32)]).reshape(NW, K, CH)

    x_pad = jnp.pad(x, ((0, NPAD - N), (0, 0)))
    zeros = jnp.zeros((ROWS_PER_TILE, D), jnp.float32)
    Wr_pad = jnp.pad(Wr, ((0, 0), (0, D - OUT)))
    br_pad = jnp.pad(br, (0, D - OUT)).reshape(1, D)

    agg = _sc_aggregate()
    acc1 = agg(x_pad, srcs, dsts, zeros)
    h1 = _tc_layer(acc1, W1, b1)
    acc2 = agg(h1, srcs, dsts, zeros)
    h2 = _tc_layer(acc2, W2, b2)
    acc3 = agg(h2, srcs, dsts, zeros)
    g = _tc_final(acc3, W3, b3, Wr_pad, br_pad)
    return g[0, :OUT]


# speedup vs baseline: 3.8978x; 1.1447x over previous
import functools

import jax
import jax.numpy as jnp
from jax import lax
from jax.experimental import pallas as pl
from jax.experimental.pallas import tpu as pltpu
from jax.experimental.pallas import tpu_sc as plsc

N = 10000
D = 128
E = 320000
OUT = 10

NC = 2
NS = 16
NW = NC * NS

CH = 128
CTOT = 2560
EPAD = CTOT * CH
K0 = 32
K1 = (CTOT - NS * K0) // NS
NPAD = 10240
ROWS_PER_TILE = NPAD // NS

def _sc_aggregate_body(h_hbm, srcs_hbm, dsts_hbm, zeros_hbm, out_hbm,
                       buf0, buf1, sidx0, sidx1, didx, acc,
                       sem0, sem1, isem0, isem1, dsem0, dsem1):
    cid = lax.axis_index("c")
    sid = lax.axis_index("s")

    kc = jnp.where(cid == 0, K0, K1)
    base = jnp.where(cid == 0, sid * K0, NS * K0 + sid * K1)

    pltpu.sync_copy(zeros_hbm, acc.at[pl.ds(sid * ROWS_PER_TILE, ROWS_PER_TILE)])
    plsc.subcore_barrier()

    bufs = (buf0, buf1)
    sems = (sem0, sem1)
    sidxs = (sidx0, sidx1)
    isems = (isem0, isem1)
    dsems = (dsem0, dsem1)

    for b in range(2):
        pltpu.async_copy(dsts_hbm.at[base + b], didx.at[b], dsems[b])
        pltpu.sync_copy(srcs_hbm.at[base + b], sidxs[b])
        pltpu.async_copy(h_hbm.at[sidxs[b]], bufs[b], sems[b])

    def body(i, carry):
        j0 = i * 2
        for b in range(2):
            j = j0 + b
            pltpu.make_async_copy(h_hbm.at[sidxs[b]], bufs[b], sems[b]).wait()

            @pl.when(j + 2 < kc)
            def _():
                pltpu.async_copy(srcs_hbm.at[base + j + 2], sidxs[b], isems[b])

            pltpu.make_async_copy(
                dsts_hbm.at[base + j], didx.at[b], dsems[b]).wait()
            pltpu.sync_copy(bufs[b], acc.at[didx.at[b]], add=True)

            @pl.when(j + 2 < kc)
            def _():
                pltpu.async_copy(dsts_hbm.at[base + j + 2], didx.at[b], dsems[b])
                pltpu.make_async_copy(
                    srcs_hbm.at[base + j + 2], sidxs[b], isems[b]).wait()
                pltpu.async_copy(h_hbm.at[sidxs[b]], bufs[b], sems[b])
        return carry

    lax.fori_loop(0, kc // 2, body, 0)

    plsc.subcore_barrier()
    pltpu.sync_copy(acc.at[pl.ds(sid * ROWS_PER_TILE, ROWS_PER_TILE)],
                    out_hbm.at[cid, pl.ds(sid * ROWS_PER_TILE, ROWS_PER_TILE)])


@functools.cache
def _sc_aggregate():
    mesh = plsc.VectorSubcoreMesh(core_axis_name="c", subcore_axis_name="s")
    return pl.kernel(
        _sc_aggregate_body,
        mesh=mesh,
        out_type=jax.ShapeDtypeStruct((NC, NPAD, D), jnp.float32),
        scratch_types=[
            pltpu.VMEM((CH, D), jnp.float32),
            pltpu.VMEM((CH, D), jnp.float32),
            pltpu.VMEM((CH,), jnp.int32),
            pltpu.VMEM((CH,), jnp.int32),
            pltpu.VMEM((2, CH), jnp.int32),
            pltpu.VMEM_SHARED((NPAD, D), jnp.float32),
            pltpu.SemaphoreType.DMA,
            pltpu.SemaphoreType.DMA,
            pltpu.SemaphoreType.DMA,
            pltpu.SemaphoreType.DMA,
            pltpu.SemaphoreType.DMA,
            pltpu.SemaphoreType.DMA,
        ],
    )


_BR = 1024


def _layer_body(acc_ref, w_ref, b_ref, o_ref):
    s = acc_ref[0] + acc_ref[1]
    o_ref[...] = jnp.tanh(
        jnp.dot(s, w_ref[...], preferred_element_type=jnp.float32) + b_ref[...])


def _tc_layer(acc, W, b):
    return pl.pallas_call(
        _layer_body,
        grid=(NPAD // _BR,),
        in_specs=[
            pl.BlockSpec((NC, _BR, D), lambda i: (0, i, 0)),
            pl.BlockSpec((D, D), lambda i: (0, 0)),
            pl.BlockSpec((1, D), lambda i: (0, 0)),
        ],
        out_specs=pl.BlockSpec((_BR, D), lambda i: (i, 0)),
        out_shape=jax.ShapeDtypeStruct((NPAD, D), jnp.float32),
    )(acc, W, b.reshape(1, D))


def _final_body(acc_ref, w_ref, b_ref, wr_ref, br_ref, o_ref, sum_ref):
    i = pl.program_id(0)

    @pl.when(i == 0)
    def _():
        sum_ref[...] = jnp.zeros_like(sum_ref)

    s = acc_ref[0] + acc_ref[1]
    t = jnp.tanh(
        jnp.dot(s, w_ref[...], preferred_element_type=jnp.float32) + b_ref[...])
    rows = lax.broadcasted_iota(jnp.int32, (_BR, 1), 0) + i * _BR
    t = jnp.where(rows < N, t, 0.0)
    sum_ref[...] += jnp.sum(t, axis=0, keepdims=True)
    m = sum_ref[...] * (1.0 / N)
    o_ref[...] = jnp.tanh(
        jnp.dot(m, wr_ref[...], preferred_element_type=jnp.float32) + br_ref[...])


def _tc_final(acc, W, b, Wr_pad, br_pad):
    return pl.pallas_call(
        _final_body,
        grid=(NPAD // _BR,),
        in_specs=[
            pl.BlockSpec((NC, _BR, D), lambda i: (0, i, 0)),
            pl.BlockSpec((D, D), lambda i: (0, 0)),
            pl.BlockSpec((1, D), lambda i: (0, 0)),
            pl.BlockSpec((D, D), lambda i: (0, 0)),
            pl.BlockSpec((1, D), lambda i: (0, 0)),
        ],
        out_specs=pl.BlockSpec((1, D), lambda i: (0, 0)),
        out_shape=jax.ShapeDtypeStruct((1, D), jnp.float32),
        scratch_shapes=[pltpu.VMEM((1, D), jnp.float32)],
    )(acc, W, b.reshape(1, D), Wr_pad, br_pad)


def kernel(x, edge_index, W1, b1, W2, b2, W3, b3, Wr, br, pos):
    del pos
    src = edge_index[0]
    dst = edge_index[1]
    pad_e = EPAD - E
    srcs = jnp.concatenate([src, jnp.zeros((pad_e,), jnp.int32)]).reshape(CTOT, CH)
    dsts = jnp.concatenate([dst, jnp.full((pad_e,), N, jnp.int32)]).reshape(CTOT, CH)

    x_pad = jnp.pad(x, ((0, NPAD - N), (0, 0)))
    zeros = jnp.zeros((ROWS_PER_TILE, D), jnp.float32)
    Wr_pad = jnp.pad(Wr, ((0, 0), (0, D - OUT)))
    br_pad = jnp.pad(br, (0, D - OUT)).reshape(1, D)

    agg = _sc_aggregate()
    acc1 = agg(x_pad, srcs, dsts, zeros)
    h1 = _tc_layer(acc1, W1, b1)
    acc2 = agg(h1, srcs, dsts, zeros)
    h2 = _tc_layer(acc2, W2, b2)
    acc3 = agg(h2, srcs, dsts, zeros)
    g = _tc_final(acc3, W3, b3, Wr_pad, br_pad)
    return g[0, :OUT]


# speedup vs baseline: 12.7895x; 3.2812x over previous
import functools

import jax
import jax.numpy as jnp
from jax import lax
from jax.experimental import pallas as pl
from jax.experimental.pallas import tpu as pltpu
from jax.experimental.pallas import tpu_sc as plsc

N = 10000
D = 128
E = 320000
OUT = 10

NC = 2
NS = 16
NW = NC * NS

CH = 128
CTOT = 2560
EPAD = CTOT * CH
K0 = 80
K1 = (CTOT - NS * K0) // NS
NPAD = 10240
ROWS_PER_TILE = NPAD // NS

def _sc_aggregate_body(h_hbm, srcs_hbm, dsts_hbm, zeros_hbm, out_hbm,
                       buf0, buf1, sidx0, sidx1, didx, acc,
                       sem0, sem1, isem0, isem1, dsem0, dsem1):
    cid = lax.axis_index("c")
    sid = lax.axis_index("s")

    kc = jnp.where(cid == 0, K0, K1)
    base = jnp.where(cid == 0, sid * K0, NS * K0 + sid * K1)

    pltpu.sync_copy(zeros_hbm, acc.at[pl.ds(sid * ROWS_PER_TILE, ROWS_PER_TILE)])
    plsc.subcore_barrier()

    bufs = (buf0, buf1)
    sems = (sem0, sem1)
    sidxs = (sidx0, sidx1)
    isems = (isem0, isem1)
    dsems = (dsem0, dsem1)

    for b in range(2):
        pltpu.async_copy(dsts_hbm.at[base + b], didx.at[b], dsems[b])
        pltpu.sync_copy(srcs_hbm.at[base + b], sidxs[b])
        pltpu.async_copy(h_hbm.at[sidxs[b]], bufs[b], sems[b])

    def body(i, carry):
        j0 = i * 2
        for b in range(2):
            j = j0 + b
            pltpu.make_async_copy(h_hbm.at[sidxs[b]], bufs[b], sems[b]).wait()

            @pl.when(j + 2 < kc)
            def _():
                pltpu.async_copy(srcs_hbm.at[base + j + 2], sidxs[b], isems[b])

            pltpu.make_async_copy(
                dsts_hbm.at[base + j], didx.at[b], dsems[b]).wait()
            pltpu.sync_copy(bufs[b], acc.at[didx.at[b]], add=True)

            @pl.when(j + 2 < kc)
            def _():
                pltpu.async_copy(dsts_hbm.at[base + j + 2], didx.at[b], dsems[b])
                pltpu.make_async_copy(
                    srcs_hbm.at[base + j + 2], sidxs[b], isems[b]).wait()
                pltpu.async_copy(h_hbm.at[sidxs[b]], bufs[b], sems[b])
        return carry

    lax.fori_loop(0, kc // 2, body, 0)

    plsc.subcore_barrier()
    pltpu.sync_copy(acc.at[pl.ds(sid * ROWS_PER_TILE, ROWS_PER_TILE)],
                    out_hbm.at[cid, pl.ds(sid * ROWS_PER_TILE, ROWS_PER_TILE)])


@functools.cache
def _sc_aggregate():
    mesh = plsc.VectorSubcoreMesh(core_axis_name="c", subcore_axis_name="s")
    return pl.kernel(
        _sc_aggregate_body,
        mesh=mesh,
        out_type=jax.ShapeDtypeStruct((NC, NPAD, D), jnp.float32),
        scratch_types=[
            pltpu.VMEM((CH, D), jnp.float32),
            pltpu.VMEM((CH, D), jnp.float32),
            pltpu.VMEM((CH,), jnp.int32),
            pltpu.VMEM((CH,), jnp.int32),
            pltpu.VMEM((2, CH), jnp.int32),
            pltpu.VMEM_SHARED((NPAD, D), jnp.float32),
            pltpu.SemaphoreType.DMA,
            pltpu.SemaphoreType.DMA,
            pltpu.SemaphoreType.DMA,
            pltpu.SemaphoreType.DMA,
            pltpu.SemaphoreType.DMA,
            pltpu.SemaphoreType.DMA,
        ],
    )


_BR = 1024


def _layer_body(acc_ref, w_ref, b_ref, o_ref):
    s = acc_ref[0] + acc_ref[1]
    o_ref[...] = jnp.tanh(
        jnp.dot(s, w_ref[...], preferred_element_type=jnp.float32) + b_ref[...])


def _tc_layer(acc, W, b):
    return pl.pallas_call(
        _layer_body,
        grid=(NPAD // _BR,),
        in_specs=[
            pl.BlockSpec((NC, _BR, D), lambda i: (0, i, 0)),
            pl.BlockSpec((D, D), lambda i: (0, 0)),
            pl.BlockSpec((1, D), lambda i: (0, 0)),
        ],
        out_specs=pl.BlockSpec((_BR, D), lambda i: (i, 0)),
        out_shape=jax.ShapeDtypeStruct((NPAD, D), jnp.float32),
    )(acc, W, b.reshape(1, D))


def _final_body(acc_ref, w_ref, b_ref, wr_ref, br_ref, o_ref, sum_ref):
    i = pl.program_id(0)

    @pl.when(i == 0)
    def _():
        sum_ref[...] = jnp.zeros_like(sum_ref)

    s = acc_ref[0] + acc_ref[1]
    t = jnp.tanh(
        jnp.dot(s, w_ref[...], preferred_element_type=jnp.float32) + b_ref[...])
    rows = lax.broadcasted_iota(jnp.int32, (_BR, 1), 0) + i * _BR
    t = jnp.where(rows < N, t, 0.0)
    sum_ref[...] += jnp.sum(t, axis=0, keepdims=True)
    m = sum_ref[...] * (1.0 / N)
    o_ref[...] = jnp.tanh(
        jnp.dot(m, wr_ref[...], preferred_element_type=jnp.float32) + br_ref[...])


def _tc_final(acc, W, b, Wr_pad, br_pad):
    return pl.pallas_call(
        _final_body,
        grid=(NPAD // _BR,),
        in_specs=[
            pl.BlockSpec((NC, _BR, D), lambda i: (0, i, 0)),
            pl.BlockSpec((D, D), lambda i: (0, 0)),
            pl.BlockSpec((1, D), lambda i: (0, 0)),
            pl.BlockSpec((D, D), lambda i: (0, 0)),
            pl.BlockSpec((1, D), lambda i: (0, 0)),
        ],
        out_specs=pl.BlockSpec((1, D), lambda i: (0, 0)),
        out_shape=jax.ShapeDtypeStruct((1, D), jnp.float32),
        scratch_shapes=[pltpu.VMEM((1, D), jnp.float32)],
    )(acc, W, b.reshape(1, D), Wr_pad, br_pad)


def kernel(x, edge_index, W1, b1, W2, b2, W3, b3, Wr, br, pos):
    del pos
    src = edge_index[0]
    dst = edge_index[1]
    pad_e = EPAD - E
    pad_i = jnp.arange(pad_e, dtype=jnp.int32)
    srcs = jnp.concatenate([src, pad_i % N]).reshape(CTOT, CH)
    dsts = jnp.concatenate([dst, N + pad_i % (NPAD - N)]).reshape(CTOT, CH)

    x_pad = jnp.pad(x, ((0, NPAD - N), (0, 0)))
    zeros = jnp.zeros((ROWS_PER_TILE, D), jnp.float32)
    Wr_pad = jnp.pad(Wr, ((0, 0), (0, D - OUT)))
    br_pad = jnp.pad(br, (0, D - OUT)).reshape(1, D)

    agg = _sc_aggregate()
    acc1 = agg(x_pad, srcs, dsts, zeros)
    h1 = _tc_layer(acc1, W1, b1)
    acc2 = agg(h1, srcs, dsts, zeros)
    h2 = _tc_layer(acc2, W2, b2)
    acc3 = agg(h2, srcs, dsts, zeros)
    g = _tc_final(acc3, W3, b3, Wr_pad, br_pad)
    return g[0, :OUT]


# speedup vs baseline: 13.3409x; 1.0431x over previous
import functools

import jax
import jax.numpy as jnp
from jax import lax
from jax.experimental import pallas as pl
from jax.experimental.pallas import tpu as pltpu
from jax.experimental.pallas import tpu_sc as plsc

N = 10000
D = 128
E = 320000
OUT = 10

NC = 2
NS = 16
NW = NC * NS

CH = 128
CTOT = 2560
CREAL = E // CH
K0 = 80
K1 = (CTOT - NS * K0) // NS
NPAD = 10240
ROWS_PER_TILE = NPAD // NS


def _start_idx_load(edge_hbm, pads_hbm, row, J, dstbuf, sem):
    @pl.when(J < CREAL)
    def _():
        pltpu.async_copy(edge_hbm.at[row, pl.ds(J * CH, CH)], dstbuf, sem)

    @pl.when(J >= CREAL)
    def _():
        pltpu.async_copy(pads_hbm.at[row, J - CREAL], dstbuf, sem)


def _sc_aggregate_body(h_hbm, edge_hbm, pads_hbm, zeros_hbm, out_hbm,
                       buf0, buf1, sidx0, sidx1, didx, acc,
                       sem0, sem1, isem0, isem1, dsem0, dsem1):
    cid = lax.axis_index("c")
    sid = lax.axis_index("s")

    kc = jnp.where(cid == 0, K0, K1)
    base = jnp.where(cid == 0, sid * K0, NS * K0 + sid * K1)

    pltpu.sync_copy(zeros_hbm, acc.at[pl.ds(sid * ROWS_PER_TILE, ROWS_PER_TILE)])
    plsc.subcore_barrier()

    bufs = (buf0, buf1)
    sems = (sem0, sem1)
    sidxs = (sidx0, sidx1)
    isems = (isem0, isem1)
    dsems = (dsem0, dsem1)

    def wait_idx(dstbuf, sem):
        pltpu.make_async_copy(pads_hbm.at[0, 0], dstbuf, sem).wait()

    for b in range(2):
        _start_idx_load(edge_hbm, pads_hbm, 1, base + b, didx.at[b], dsems[b])
        _start_idx_load(edge_hbm, pads_hbm, 0, base + b, sidxs[b], isems[b])
        wait_idx(sidxs[b], isems[b])
        pltpu.async_copy(h_hbm.at[sidxs[b]], bufs[b], sems[b])

    def body(i, carry):
        j0 = i * 2
        for b in range(2):
            j = j0 + b
            pltpu.make_async_copy(h_hbm.at[sidxs[b]], bufs[b], sems[b]).wait()

            @pl.when(j + 2 < kc)
            def _():
                _start_idx_load(edge_hbm, pads_hbm, 0, base + j + 2,
                                sidxs[b], isems[b])

            wait_idx(didx.at[b], dsems[b])
            pltpu.sync_copy(bufs[b], acc.at[didx.at[b]], add=True)

            @pl.when(j + 2 < kc)
            def _():
                _start_idx_load(edge_hbm, pads_hbm, 1, base + j + 2,
                                didx.at[b], dsems[b])
                wait_idx(sidxs[b], isems[b])
                pltpu.async_copy(h_hbm.at[sidxs[b]], bufs[b], sems[b])
        return carry

    lax.fori_loop(0, kc // 2, body, 0)

    plsc.subcore_barrier()
    pltpu.sync_copy(acc.at[pl.ds(sid * ROWS_PER_TILE, ROWS_PER_TILE)],
                    out_hbm.at[cid, pl.ds(sid * ROWS_PER_TILE, ROWS_PER_TILE)])


@functools.cache
def _sc_aggregate():
    mesh = plsc.VectorSubcoreMesh(core_axis_name="c", subcore_axis_name="s")
    return pl.kernel(
        _sc_aggregate_body,
        mesh=mesh,
        out_type=jax.ShapeDtypeStruct((NC, NPAD, D), jnp.float32),
        scratch_types=[
            pltpu.VMEM((CH, D), jnp.float32),
            pltpu.VMEM((CH, D), jnp.float32),
            pltpu.VMEM((CH,), jnp.int32),
            pltpu.VMEM((CH,), jnp.int32),
            pltpu.VMEM((2, CH), jnp.int32),
            pltpu.VMEM_SHARED((NPAD, D), jnp.float32),
            pltpu.SemaphoreType.DMA,
            pltpu.SemaphoreType.DMA,
            pltpu.SemaphoreType.DMA,
            pltpu.SemaphoreType.DMA,
            pltpu.SemaphoreType.DMA,
            pltpu.SemaphoreType.DMA,
        ],
    )


_BR = 1000


def _layer_body(acc_ref, w_ref, b_ref, o_ref):
    s = acc_ref[0] + acc_ref[1]
    o_ref[...] = jnp.tanh(
        jnp.dot(s, w_ref[...], preferred_element_type=jnp.float32) + b_ref[...])


def _tc_layer(acc, W, b):
    return pl.pallas_call(
        _layer_body,
        grid=(N // _BR,),
        in_specs=[
            pl.BlockSpec((NC, _BR, D), lambda i: (0, i, 0)),
            pl.BlockSpec((D, D), lambda i: (0, 0)),
            pl.BlockSpec((1, D), lambda i: (0, 0)),
        ],
        out_specs=pl.BlockSpec((_BR, D), lambda i: (i, 0)),
        out_shape=jax.ShapeDtypeStruct((N, D), jnp.float32),
    )(acc, W, b.reshape(1, D))


def _final_body(acc_ref, w_ref, b_ref, wr_ref, br_ref, o_ref, sum_ref):
    i = pl.program_id(0)

    @pl.when(i == 0)
    def _():
        sum_ref[...] = jnp.zeros_like(sum_ref)

    s = acc_ref[0] + acc_ref[1]
    t = jnp.tanh(
        jnp.dot(s, w_ref[...], preferred_element_type=jnp.float32) + b_ref[...])
    sum_ref[...] += jnp.sum(t, axis=0, keepdims=True)
    m = sum_ref[...] * (1.0 / N)
    o_ref[...] = jnp.tanh(
        jnp.dot(m, wr_ref[...], preferred_element_type=jnp.float32) + br_ref[...])


def _tc_final(acc, W, b, Wr_pad, br_pad):
    return pl.pallas_call(
        _final_body,
        grid=(N // _BR,),
        in_specs=[
            pl.BlockSpec((NC, _BR, D), lambda i: (0, i, 0)),
            pl.BlockSpec((D, D), lambda i: (0, 0)),
            pl.BlockSpec((1, D), lambda i: (0, 0)),
            pl.BlockSpec((D, D), lambda i: (0, 0)),
            pl.BlockSpec((1, D), lambda i: (0, 0)),
        ],
        out_specs=pl.BlockSpec((1, D), lambda i: (0, 0)),
        out_shape=jax.ShapeDtypeStruct((1, D), jnp.float32),
        scratch_shapes=[pltpu.VMEM((1, D), jnp.float32)],
    )(acc, W, b.reshape(1, D), Wr_pad, br_pad)


def kernel(x, edge_index, W1, b1, W2, b2, W3, b3, Wr, br, pos):
    del pos
    pad_e = (CTOT - CREAL) * CH
    pad_i = jnp.arange(pad_e, dtype=jnp.int32)
    pads = jnp.stack([pad_i % N, N + pad_i % (NPAD - N)]).reshape(
        2, CTOT - CREAL, CH)

    zeros = jnp.zeros((ROWS_PER_TILE, D), jnp.float32)
    Wr_pad = jnp.pad(Wr, ((0, 0), (0, D - OUT)))
    br_pad = jnp.pad(br, (0, D - OUT)).reshape(1, D)

    agg = _sc_aggregate()
    acc1 = agg(x, edge_index, pads, zeros)
    h1 = _tc_layer(acc1, W1, b1)
    acc2 = agg(h1, edge_index, pads, zeros)
    h2 = _tc_layer(acc2, W2, b2)
    acc3 = agg(h2, edge_index, pads, zeros)
    g = _tc_final(acc3, W3, b3, Wr_pad, br_pad)
    return g[0, :OUT]


# speedup vs baseline: 14.7998x; 1.1094x over previous
import functools

import jax
import jax.numpy as jnp
from jax import lax
from jax.experimental import pallas as pl
from jax.experimental.pallas import tpu as pltpu
from jax.experimental.pallas import tpu_sc as plsc

N = 10000
D = 128
E = 320000
OUT = 10

NC = 2
NS = 16
NW = NC * NS

CH = 128
CTOT = 2592
CREAL = E // CH
K0 = 81
K1 = (CTOT - NS * K0) // NS
NPAD = 10112
ROWS_PER_TILE = NPAD // NS
RING = 3


def _start_idx_load(edge_hbm, pads_hbm, row, J, dstbuf, sem):
    @pl.when(J < CREAL)
    def _():
        pltpu.async_copy(edge_hbm.at[row, pl.ds(J * CH, CH)], dstbuf, sem)

    @pl.when(J >= CREAL)
    def _():
        pltpu.async_copy(pads_hbm.at[row, J - CREAL], dstbuf, sem)


def _sc_aggregate_body(h_hbm, edge_hbm, pads_hbm, zeros_hbm, out_hbm,
                       bufs, sidxs, didx, acc, sems, isems, dsems):
    cid = lax.axis_index("c")
    sid = lax.axis_index("s")

    kc = jnp.where(cid == 0, K0, K1)
    base = jnp.where(cid == 0, sid * K0, NS * K0 + sid * K1)

    def wait_idx(dstbuf, sem):
        pltpu.make_async_copy(pads_hbm.at[0, 0], dstbuf, sem).wait()

    for b in range(RING):
        _start_idx_load(edge_hbm, pads_hbm, 1, base + b, didx.at[b], dsems[b])
        _start_idx_load(edge_hbm, pads_hbm, 0, base + b, sidxs[b], isems[b])
    for b in range(RING):
        wait_idx(sidxs[b], isems[b])
        pltpu.async_copy(h_hbm.at[sidxs[b]], bufs[b], sems[b])
    pltpu.sync_copy(zeros_hbm, acc.at[pl.ds(sid * ROWS_PER_TILE, ROWS_PER_TILE)])
    plsc.subcore_barrier()

    def body(i, carry):
        j0 = i * RING
        for b in range(RING):
            j = j0 + b
            pltpu.make_async_copy(h_hbm.at[sidxs[b]], bufs[b], sems[b]).wait()

            @pl.when(j + RING < kc)
            def _():
                _start_idx_load(edge_hbm, pads_hbm, 0, base + j + RING,
                                sidxs[b], isems[b])

            wait_idx(didx.at[b], dsems[b])
            pltpu.sync_copy(bufs[b], acc.at[didx.at[b]], add=True)

            @pl.when(j + RING < kc)
            def _():
                _start_idx_load(edge_hbm, pads_hbm, 1, base + j + RING,
                                didx.at[b], dsems[b])
                wait_idx(sidxs[b], isems[b])
                pltpu.async_copy(h_hbm.at[sidxs[b]], bufs[b], sems[b])
        return carry

    lax.fori_loop(0, kc // RING, body, 0)

    plsc.subcore_barrier()
    pltpu.sync_copy(acc.at[pl.ds(sid * ROWS_PER_TILE, ROWS_PER_TILE)],
                    out_hbm.at[cid, pl.ds(sid * ROWS_PER_TILE, ROWS_PER_TILE)])


@functools.cache
def _sc_aggregate():
    mesh = plsc.VectorSubcoreMesh(core_axis_name="c", subcore_axis_name="s")
    return pl.kernel(
        _sc_aggregate_body,
        mesh=mesh,
        out_type=jax.ShapeDtypeStruct((NC, NPAD, D), jnp.float32),
        scratch_types=[
            [pltpu.VMEM((CH, D), jnp.float32) for _ in range(RING)],
            [pltpu.VMEM((CH,), jnp.int32) for _ in range(RING)],
            pltpu.VMEM((RING, CH), jnp.int32),
            pltpu.VMEM_SHARED((NPAD, D), jnp.float32),
            [pltpu.SemaphoreType.DMA for _ in range(RING)],
            [pltpu.SemaphoreType.DMA for _ in range(RING)],
            [pltpu.SemaphoreType.DMA for _ in range(RING)],
        ],
    )


_BR = 1000


def _layer_body(acc_ref, w_ref, b_ref, o_ref):
    s = acc_ref[0] + acc_ref[1]
    o_ref[...] = jnp.tanh(
        jnp.dot(s, w_ref[...], preferred_element_type=jnp.float32) + b_ref[...])


def _tc_layer(acc, W, b):
    return pl.pallas_call(
        _layer_body,
        grid=(N // _BR,),
        in_specs=[
            pl.BlockSpec((NC, _BR, D), lambda i: (0, i, 0)),
            pl.BlockSpec((D, D), lambda i: (0, 0)),
            pl.BlockSpec((1, D), lambda i: (0, 0)),
        ],
        out_specs=pl.BlockSpec((_BR, D), lambda i: (i, 0)),
        out_shape=jax.ShapeDtypeStruct((N, D), jnp.float32),
    )(acc, W, b.reshape(1, D))


def _final_body(acc_ref, w_ref, b_ref, wr_ref, br_ref, o_ref, sum_ref):
    i = pl.program_id(0)

    @pl.when(i == 0)
    def _():
        sum_ref[...] = jnp.zeros_like(sum_ref)

    s = acc_ref[0] + acc_ref[1]
    t = jnp.tanh(
        jnp.dot(s, w_ref[...], preferred_element_type=jnp.float32) + b_ref[...])
    sum_ref[...] += jnp.sum(t, axis=0, keepdims=True)
    m = sum_ref[...] * (1.0 / N)
    o_ref[...] = jnp.tanh(
        jnp.dot(m, wr_ref[...], preferred_element_type=jnp.float32) + br_ref[...])


def _tc_final(acc, W, b, Wr_pad, br_pad):
    return pl.pallas_call(
        _final_body,
        grid=(N // _BR,),
        in_specs=[
            pl.BlockSpec((NC, _BR, D), lambda i: (0, i, 0)),
            pl.BlockSpec((D, D), lambda i: (0, 0)),
            pl.BlockSpec((1, D), lambda i: (0, 0)),
            pl.BlockSpec((D, D), lambda i: (0, 0)),
            pl.BlockSpec((1, D), lambda i: (0, 0)),
        ],
        out_specs=pl.BlockSpec((1, D), lambda i: (0, 0)),
        out_shape=jax.ShapeDtypeStruct((1, D), jnp.float32),
        scratch_shapes=[pltpu.VMEM((1, D), jnp.float32)],
    )(acc, W, b.reshape(1, D), Wr_pad, br_pad)


def kernel(x, edge_index, W1, b1, W2, b2, W3, b3, Wr, br, pos):
    del pos
    pad_e = (CTOT - CREAL) * CH
    pad_i = jnp.arange(pad_e, dtype=jnp.int32)
    pads = jnp.stack([pad_i % N, N + pad_i % (NPAD - N)]).reshape(
        2, CTOT - CREAL, CH)

    zeros = jnp.zeros((ROWS_PER_TILE, D), jnp.float32)
    Wr_pad = jnp.pad(Wr, ((0, 0), (0, D - OUT)))
    br_pad = jnp.pad(br, (0, D - OUT)).reshape(1, D)

    agg = _sc_aggregate()
    acc1 = agg(x, edge_index, pads, zeros)
    h1 = _tc_layer(acc1, W1, b1)
    acc2 = agg(h1, edge_index, pads, zeros)
    h2 = _tc_layer(acc2, W2, b2)
    acc3 = agg(h2, edge_index, pads, zeros)
    g = _tc_final(acc3, W3, b3, Wr_pad, br_pad)
    return g[0, :OUT]


# speedup vs baseline: 15.1408x; 1.0230x over previous
import functools

import jax
import jax.numpy as jnp
from jax import lax
from jax.experimental import pallas as pl
from jax.experimental.pallas import tpu as pltpu
from jax.experimental.pallas import tpu_sc as plsc

N = 10000
D = 128
E = 320000
OUT = 10

NC = 2
NS = 16
NW = NC * NS

CH = 128
CTOT = 2592
CREAL = E // CH
K0 = 81
K1 = (CTOT - NS * K0) // NS
NPAD = 10112
ROWS_PER_TILE = NPAD // NS
RING = 3


def _start_idx_load(edge_hbm, pads_hbm, row, J, dstbuf, sem):
    @pl.when(J < CREAL)
    def _():
        pltpu.async_copy(edge_hbm.at[row, pl.ds(J * CH, CH)], dstbuf, sem)

    @pl.when(J >= CREAL)
    def _():
        pltpu.async_copy(pads_hbm.at[row, J - CREAL], dstbuf, sem)


def _sc_aggregate_body(h_hbm, edge_hbm, pads_hbm, zeros_hbm, out_hbm,
                       bufs, sidxs, didx, acc, sems, isems, dsems):
    cid = lax.axis_index("c")
    sid = lax.axis_index("s")

    kc = jnp.where(cid == 0, K0, K1)
    base = jnp.where(cid == 0, sid * K0, NS * K0 + sid * K1)

    def wait_idx(dstbuf, sem):
        pltpu.make_async_copy(pads_hbm.at[0, 0], dstbuf, sem).wait()

    for b in range(RING):
        _start_idx_load(edge_hbm, pads_hbm, 1, base + b, didx.at[b], dsems[b])
        _start_idx_load(edge_hbm, pads_hbm, 0, base + b, sidxs[b], isems[b])
    for b in range(RING):
        wait_idx(sidxs[b], isems[b])
        pltpu.async_copy(h_hbm.at[sidxs[b]], bufs[b], sems[b])
    pltpu.sync_copy(zeros_hbm, acc.at[pl.ds(sid * ROWS_PER_TILE, ROWS_PER_TILE)])
    plsc.subcore_barrier()

    def body(i, carry):
        j0 = i * RING
        for b in range(RING):
            j = j0 + b
            pltpu.make_async_copy(h_hbm.at[sidxs[b]], bufs[b], sems[b]).wait()

            @pl.when(j + RING < kc)
            def _():
                _start_idx_load(edge_hbm, pads_hbm, 0, base + j + RING,
                                sidxs[b], isems[b])

            wait_idx(didx.at[b], dsems[b])
            pltpu.sync_copy(bufs[b], acc.at[didx.at[b]], add=True)

            @pl.when(j + RING < kc)
            def _():
                _start_idx_load(edge_hbm, pads_hbm, 1, base + j + RING,
                                didx.at[b], dsems[b])
                wait_idx(sidxs[b], isems[b])
                pltpu.async_copy(h_hbm.at[sidxs[b]], bufs[b], sems[b])
        return carry

    lax.fori_loop(0, kc // RING, body, 0)

    plsc.subcore_barrier()
    pltpu.sync_copy(acc.at[pl.ds(sid * ROWS_PER_TILE, ROWS_PER_TILE)],
                    out_hbm.at[cid, pl.ds(sid * ROWS_PER_TILE, ROWS_PER_TILE)])


@functools.cache
def _sc_aggregate():
    mesh = plsc.VectorSubcoreMesh(core_axis_name="c", subcore_axis_name="s")
    return pl.kernel(
        _sc_aggregate_body,
        mesh=mesh,
        out_type=jax.ShapeDtypeStruct((NC, NPAD, D), jnp.float32),
        scratch_types=[
            [pltpu.VMEM((CH, D), jnp.float32) for _ in range(RING)],
            [pltpu.VMEM((CH,), jnp.int32) for _ in range(RING)],
            pltpu.VMEM((RING, CH), jnp.int32),
            pltpu.VMEM_SHARED((NPAD, D), jnp.float32),
            [pltpu.SemaphoreType.DMA for _ in range(RING)],
            [pltpu.SemaphoreType.DMA for _ in range(RING)],
            [pltpu.SemaphoreType.DMA for _ in range(RING)],
        ],
    )


_BR = 2000


def _layer_body(acc_ref, w_ref, b_ref, o_ref):
    s = acc_ref[0] + acc_ref[1]
    o_ref[...] = jnp.tanh(
        jnp.dot(s, w_ref[...], preferred_element_type=jnp.float32) + b_ref[...])


def _tc_layer(acc, W, b):
    return pl.pallas_call(
        _layer_body,
        grid=(N // _BR,),
        in_specs=[
            pl.BlockSpec((NC, _BR, D), lambda i: (0, i, 0)),
            pl.BlockSpec((D, D), lambda i: (0, 0)),
            pl.BlockSpec((1, D), lambda i: (0, 0)),
        ],
        out_specs=pl.BlockSpec((_BR, D), lambda i: (i, 0)),
        out_shape=jax.ShapeDtypeStruct((N, D), jnp.float32),
    )(acc, W, b.reshape(1, D))


def _final_body(acc_ref, w_ref, b_ref, wr_ref, br_ref, o_ref, sum_ref):
    i = pl.program_id(0)

    @pl.when(i == 0)
    def _():
        sum_ref[...] = jnp.zeros_like(sum_ref)

    s = acc_ref[0] + acc_ref[1]
    t = jnp.tanh(
        jnp.dot(s, w_ref[...], preferred_element_type=jnp.float32) + b_ref[...])
    sum_ref[...] += jnp.sum(t, axis=0, keepdims=True)
    m = sum_ref[...] * (1.0 / N)
    o_ref[...] = jnp.tanh(
        jnp.dot(m, wr_ref[...], preferred_element_type=jnp.float32) + br_ref[...])


def _tc_final(acc, W, b, Wr_pad, br_pad):
    return pl.pallas_call(
        _final_body,
        grid=(N // _BR,),
        in_specs=[
            pl.BlockSpec((NC, _BR, D), lambda i: (0, i, 0)),
            pl.BlockSpec((D, D), lambda i: (0, 0)),
            pl.BlockSpec((1, D), lambda i: (0, 0)),
            pl.BlockSpec((D, D), lambda i: (0, 0)),
            pl.BlockSpec((1, D), lambda i: (0, 0)),
        ],
        out_specs=pl.BlockSpec((1, D), lambda i: (0, 0)),
        out_shape=jax.ShapeDtypeStruct((1, D), jnp.float32),
        scratch_shapes=[pltpu.VMEM((1, D), jnp.float32)],
    )(acc, W, b.reshape(1, D), Wr_pad, br_pad)


def kernel(x, edge_index, W1, b1, W2, b2, W3, b3, Wr, br, pos):
    del pos
    pad_e = (CTOT - CREAL) * CH
    pad_i = jnp.arange(pad_e, dtype=jnp.int32)
    pads = jnp.stack([pad_i % N, N + pad_i % (NPAD - N)]).reshape(
        2, CTOT - CREAL, CH)

    zeros = jnp.zeros((ROWS_PER_TILE, D), jnp.float32)
    Wr_pad = jnp.pad(Wr, ((0, 0), (0, D - OUT)))
    br_pad = jnp.pad(br, (0, D - OUT)).reshape(1, D)

    agg = _sc_aggregate()
    acc1 = agg(x, edge_index, pads, zeros)
    h1 = _tc_layer(acc1, W1, b1)
    acc2 = agg(h1, edge_index, pads, zeros)
    h2 = _tc_layer(acc2, W2, b2)
    acc3 = agg(h2, edge_index, pads, zeros)
    g = _tc_final(acc3, W3, b3, Wr_pad, br_pad)
    return g[0, :OUT]
